# baseline (speedup 1.0000x reference)
import jax
import jax.numpy as jnp
from jax.experimental import pallas as pl

_N_BLK = 2000


def _mm_t_kernel(w_ref, x_ref, b_ref, o_ref):
    acc = jax.lax.dot_general(
        w_ref[...], x_ref[...],
        dimension_numbers=(((1,), (1,)), ((), ())),
        preferred_element_type=jnp.float32)
    o_ref[...] = acc + b_ref[0].T


def kernel(x, y, W, b):
    M, K = x.shape
    N = W.shape[0]
    out_t = pl.pallas_call(
        _mm_t_kernel,
        grid=(pl.cdiv(N, _N_BLK),),
        in_specs=[
            pl.BlockSpec((_N_BLK, K), lambda j: (j, 0)),
            pl.BlockSpec((M, K), lambda j: (0, 0)),
            pl.BlockSpec((1, 1, _N_BLK), lambda j: (j, 0, 0)),
        ],
        out_specs=pl.BlockSpec((_N_BLK, M), lambda j: (j, 0)),
        out_shape=jax.ShapeDtypeStruct((N, M), jnp.float32),
    )(W, x, b.reshape(N // _N_BLK, 1, _N_BLK))
    return out_t.T


# speedup vs baseline: 1.0234x; 1.0234x over previous
import jax
import jax.numpy as jnp
from jax.experimental import pallas as pl

_N_BLK = 5000


def _mm_t_kernel(w_ref, x_ref, b_ref, o_ref):
    acc = jax.lax.dot_general(
        w_ref[...], x_ref[...],
        dimension_numbers=(((1,), (1,)), ((), ())),
        preferred_element_type=jnp.float32)
    j = pl.program_id(0)
    o_ref[...] = acc + b_ref[j].T


def kernel(x, y, W, b):
    M, K = x.shape
    N = W.shape[0]
    out_t = pl.pallas_call(
        _mm_t_kernel,
        grid=(pl.cdiv(N, _N_BLK),),
        in_specs=[
            pl.BlockSpec((_N_BLK, K), lambda j: (j, 0)),
            pl.BlockSpec((M, K), lambda j: (0, 0)),
            pl.BlockSpec((N // _N_BLK, 1, _N_BLK), lambda j: (0, 0, 0)),
        ],
        out_specs=pl.BlockSpec((_N_BLK, M), lambda j: (j, 0)),
        out_shape=jax.ShapeDtypeStruct((N, M), jnp.float32),
    )(W, x, b.reshape(N // _N_BLK, 1, _N_BLK))
    return out_t.T
